# speedup vs baseline: 7.1676x; 7.1676x over previous
import functools

import jax
import jax.numpy as jnp
from jax import lax
from jax.experimental import pallas as pl
from jax.experimental.pallas import tpu as pltpu
from jax.experimental.pallas import tpu_sc as plsc

D = 128
L = 128
NC = 2
NS = 16
NW = NC * NS
CG = 160


def _make_sc_gather(n_rows: int):
    n_chunks = n_rows // L
    per_w = n_chunks // NW
    ng = per_w // CG
    assert n_chunks % NW == 0 and per_w % CG == 0

    mesh = plsc.VectorSubcoreMesh(core_axis_name="c", subcore_axis_name="s")

    @functools.partial(
        pl.kernel,
        out_type=jax.ShapeDtypeStruct((n_rows, D), jnp.float32),
        mesh=mesh,
        scratch_types=[
            pltpu.VMEM((CG, L), jnp.int32),
            pltpu.VMEM((L, D), jnp.float32),
            pltpu.SemaphoreType.DMA,
        ],
    )
    def k(table_hbm, idx_hbm, out_hbm, idx_v, rows_v, sem):
        wid = lax.axis_index("s") * NC + lax.axis_index("c")
        chunk0 = wid * per_w

        def group_body(g, _):
            gchunk = chunk0 + g * CG
            pltpu.sync_copy(idx_hbm.at[pl.ds(gchunk, CG)], idx_v)

            def chunk_body(j, _):
                row0 = (gchunk + j) * L
                pltpu.async_copy(table_hbm.at[idx_v.at[j]], rows_v, sem).wait()
                pltpu.sync_copy(rows_v, out_hbm.at[pl.ds(row0, L)])
                return 0

            lax.fori_loop(0, CG, chunk_body, 0)
            return 0

        lax.fori_loop(0, ng, group_body, 0)

    return k


_gather = _make_sc_gather(16384 * 200)


@jax.jit
def kernel(x, weight):
    idx2d = x.reshape(-1, L).astype(jnp.int32)
    out = _gather(weight, idx2d)
    return out.reshape(x.shape + (D,))


# speedup vs baseline: 10.7624x; 1.5015x over previous
import functools

import jax
import jax.numpy as jnp
from jax import lax
from jax.experimental import pallas as pl
from jax.experimental.pallas import tpu as pltpu
from jax.experimental.pallas import tpu_sc as plsc

D = 128
L = 128
NC = 2
NS = 16
NW = NC * NS
CG = 160
NBUF = 5
K = 3


def _make_sc_gather(n_rows: int):
    n_chunks = n_rows // L
    per_w = n_chunks // NW
    ng = per_w // CG
    assert n_chunks % NW == 0 and per_w % CG == 0 and CG % NBUF == 0

    mesh = plsc.VectorSubcoreMesh(core_axis_name="c", subcore_axis_name="s")

    @functools.partial(
        pl.kernel,
        out_type=jax.ShapeDtypeStruct((n_rows, D), jnp.float32),
        mesh=mesh,
        scratch_types=[
            pltpu.VMEM((CG, L), jnp.int32),
            pltpu.VMEM((NBUF, L, D), jnp.float32),
        ]
        + [pltpu.SemaphoreType.DMA] * NBUF
        + [pltpu.SemaphoreType.DMA] * NBUF,
    )
    def k(table_hbm, idx_hbm, out_hbm, idx_v, rows_v, *sems):
        gsem = sems[:NBUF]
        osem = sems[NBUF:]
        wid = lax.axis_index("s") * NC + lax.axis_index("c")
        chunk0 = wid * per_w

        def start_gather(chunk_in_group, buf):
            pltpu.async_copy(
                table_hbm.at[idx_v.at[chunk_in_group]], rows_v.at[buf],
                gsem[buf])

        def group_body(g, _):
            gchunk = chunk0 + g * CG
            pltpu.sync_copy(idx_hbm.at[pl.ds(gchunk, CG)], idx_v)

            for b in range(K):
                start_gather(b, b)

            def steady(j0, _):
                for b in range(NBUF):
                    j = j0 + b
                    bk = (b + K) % NBUF

                    @pl.when(j + K < CG)
                    def _():
                        @pl.when(j >= NBUF - K)
                        def _():
                            pltpu.make_async_copy(
                                rows_v.at[bk], out_hbm.at[pl.ds(0, L)],
                                osem[bk]).wait()
                        start_gather(j + K, bk)

                    pltpu.make_async_copy(
                        table_hbm.at[pl.ds(0, L)], rows_v.at[b],
                        gsem[b]).wait()
                    pltpu.async_copy(
                        rows_v.at[b], out_hbm.at[pl.ds((gchunk + j) * L, L)],
                        osem[b])
                return 0

            lax.fori_loop(0, CG // NBUF, lambda i, c: steady(i * NBUF, c), 0)

            for b in range(NBUF):
                pltpu.make_async_copy(
                    rows_v.at[b], out_hbm.at[pl.ds(0, L)], osem[b]).wait()
            return 0

        lax.fori_loop(0, ng, group_body, 0)

    return k


_gather = _make_sc_gather(16384 * 200)


@jax.jit
def kernel(x, weight):
    idx2d = x.reshape(-1, L).astype(jnp.int32)
    out = _gather(weight, idx2d)
    return out.reshape(x.shape + (D,))


# speedup vs baseline: 18.7373x; 1.7410x over previous
import functools

import jax
import jax.numpy as jnp
from jax import lax
from jax.experimental import pallas as pl
from jax.experimental.pallas import tpu as pltpu
from jax.experimental.pallas import tpu_sc as plsc

VOCAB = 10000
D = 128
L = 128
NC = 2
NS = 16
NW = NC * NS
CG = 80
NBUF = 2
K = 1


def _make_sc_gather(n_rows: int):
    n_chunks = n_rows // L
    per_w = n_chunks // NW
    ng = per_w // CG
    assert n_chunks % NW == 0 and per_w % CG == 0 and CG % NBUF == 0

    mesh = plsc.VectorSubcoreMesh(core_axis_name="c", subcore_axis_name="s")

    @functools.partial(
        pl.kernel,
        out_type=jax.ShapeDtypeStruct((n_rows, D), jnp.float32),
        mesh=mesh,
        scratch_types=[
            pltpu.VMEM((CG, L), jnp.int32),
            pltpu.VMEM((NBUF, L, D), jnp.float32),
            pltpu.VMEM_SHARED((VOCAB, D), jnp.float32),
        ]
        + [pltpu.SemaphoreType.DMA] * NBUF
        + [pltpu.SemaphoreType.DMA] * NBUF,
    )
    def k(table_hbm, idx_hbm, out_hbm, idx_v, rows_v, table_sp, *sems):
        gsem = sems[:NBUF]
        osem = sems[NBUF:]
        sid = lax.axis_index("s")
        wid = sid * NC + lax.axis_index("c")
        chunk0 = wid * per_w

        vslice = (VOCAB // NS) // 8 * 8
        pltpu.sync_copy(table_hbm.at[pl.ds(sid * vslice, vslice)],
                        table_sp.at[pl.ds(sid * vslice, vslice)])
        tail = NS * vslice
        @pl.when(sid == NS - 1)
        def _():
            pltpu.sync_copy(table_hbm.at[pl.ds(tail, VOCAB - tail)],
                            table_sp.at[pl.ds(tail, VOCAB - tail)])
        plsc.subcore_barrier()

        def start_gather(chunk_in_group, buf):
            pltpu.async_copy(
                table_sp.at[idx_v.at[chunk_in_group]], rows_v.at[buf],
                gsem[buf])

        def group_body(g, _):
            gchunk = chunk0 + g * CG
            pltpu.sync_copy(idx_hbm.at[pl.ds(gchunk, CG)], idx_v)

            for b in range(K):
                start_gather(b, b)

            def steady(j0, _):
                for b in range(NBUF):
                    j = j0 + b
                    bk = (b + K) % NBUF

                    @pl.when(j + K < CG)
                    def _():
                        @pl.when(j >= NBUF - K)
                        def _():
                            pltpu.make_async_copy(
                                rows_v.at[bk], out_hbm.at[pl.ds(0, L)],
                                osem[bk]).wait()
                        start_gather(j + K, bk)

                    pltpu.make_async_copy(
                        table_hbm.at[pl.ds(0, L)], rows_v.at[b],
                        gsem[b]).wait()
                    pltpu.async_copy(
                        rows_v.at[b], out_hbm.at[pl.ds((gchunk + j) * L, L)],
                        osem[b])
                return 0

            lax.fori_loop(0, CG // NBUF, lambda i, c: steady(i * NBUF, c), 0)

            for b in range(NBUF):
                pltpu.make_async_copy(
                    rows_v.at[b], out_hbm.at[pl.ds(0, L)], osem[b]).wait()
            return 0

        lax.fori_loop(0, ng, group_body, 0)

    return k


_gather = _make_sc_gather(16384 * 200)


@jax.jit
def kernel(x, weight):
    idx2d = x.reshape(-1, L).astype(jnp.int32)
    out = _gather(weight, idx2d)
    return out.reshape(x.shape + (D,))


# speedup vs baseline: 19.0294x; 1.0156x over previous
import functools

import jax
import jax.numpy as jnp
from jax import lax
from jax.experimental import pallas as pl
from jax.experimental.pallas import tpu as pltpu
from jax.experimental.pallas import tpu_sc as plsc

VOCAB = 10000
D = 128
L = 128
NC = 2
NS = 16
NW = NC * NS
NBUF = 3


def _make_sc_gather(n_rows: int):
    n_chunks = n_rows // L
    per_w = n_chunks // NW
    rem = per_w % NBUF
    assert n_chunks % NW == 0 and per_w > NBUF + rem

    mesh = plsc.VectorSubcoreMesh(core_axis_name="c", subcore_axis_name="s")

    @functools.partial(
        pl.kernel,
        out_type=jax.ShapeDtypeStruct((n_rows, D), jnp.float32),
        mesh=mesh,
        scratch_types=[
            pltpu.VMEM((NBUF, 1, L), jnp.int32),
            pltpu.VMEM((NBUF, L, D), jnp.float32),
            pltpu.VMEM_SHARED((VOCAB, D), jnp.float32),
        ]
        + [pltpu.SemaphoreType.DMA] * NBUF
        + [pltpu.SemaphoreType.DMA] * NBUF
        + [pltpu.SemaphoreType.DMA] * NBUF,
    )
    def k(table_hbm, idx_hbm, out_hbm, idx_v, rows_v, table_sp, *sems):
        isem = sems[:NBUF]
        gsem = sems[NBUF:2 * NBUF]
        osem = sems[2 * NBUF:]
        sid = lax.axis_index("s")
        wid = sid * NC + lax.axis_index("c")
        chunk0 = wid * per_w

        vslice = (VOCAB // NS) // 8 * 8
        pltpu.sync_copy(table_hbm.at[pl.ds(sid * vslice, vslice)],
                        table_sp.at[pl.ds(sid * vslice, vslice)])
        tail = NS * vslice
        @pl.when(sid == NS - 1)
        def _():
            pltpu.sync_copy(table_hbm.at[pl.ds(tail, VOCAB - tail)],
                            table_sp.at[pl.ds(tail, VOCAB - tail)])
        plsc.subcore_barrier()

        def start_idx(j, b):
            pltpu.async_copy(idx_hbm.at[pl.ds(chunk0 + j, 1)], idx_v.at[b],
                             isem[b])

        def start_gather(b):
            pltpu.async_copy(table_sp.at[idx_v.at[b, 0]], rows_v.at[b],
                             gsem[b])

        def wait_idx(b):
            pltpu.make_async_copy(idx_hbm.at[pl.ds(0, 1)], idx_v.at[b],
                                  isem[b]).wait()

        def wait_gather(b):
            pltpu.make_async_copy(table_hbm.at[pl.ds(0, L)], rows_v.at[b],
                                  gsem[b]).wait()

        def wait_scatter(b):
            pltpu.make_async_copy(rows_v.at[b], out_hbm.at[pl.ds(0, L)],
                                  osem[b]).wait()

        start_idx(0, 0)
        start_idx(1, 1)
        wait_idx(0)
        start_gather(0)

        def steady(j0, _):
            for b in range(NBUF):
                j = j0 + b
                b1 = (b + 1) % NBUF
                b2 = (b + 2) % NBUF

                @pl.when(j + 2 < per_w)
                def _():
                    start_idx(j + 2, b2)

                @pl.when(j + 1 < per_w)
                def _():
                    @pl.when(j >= 2)
                    def _():
                        wait_scatter(b1)
                    wait_idx(b1)
                    start_gather(b1)

                wait_gather(b)
                pltpu.async_copy(
                    rows_v.at[b], out_hbm.at[pl.ds((chunk0 + j) * L, L)],
                    osem[b])
            return 0

        lax.fori_loop(0, (per_w - rem) // NBUF,
                      lambda i, c: steady(i * NBUF, c), 0)

        for j in range(per_w - rem, per_w):
            b = j % NBUF
            b1 = (b + 1) % NBUF
            if j + 1 < per_w:
                wait_scatter(b1)
                wait_idx(b1)
                start_gather(b1)
            wait_gather(b)
            pltpu.async_copy(
                rows_v.at[b], out_hbm.at[pl.ds((chunk0 + j) * L, L)],
                osem[b])

        for b in range(NBUF):
            wait_scatter(b)

    return k


_gather = _make_sc_gather(16384 * 200)


@jax.jit
def kernel(x, weight):
    idx2d = x.reshape(-1, L).astype(jnp.int32)
    out = _gather(weight, idx2d)
    return out.reshape(x.shape + (D,))
